# baseline (speedup 1.0000x reference)
import functools

import jax
import jax.numpy as jnp
from jax.experimental import pallas as pl
from jax.experimental.pallas import tpu as pltpu

VOCAB_ = 100000
EMB_ = 128
CTX2_ = 50
HID_ = 128
VTILE_ = 18432
NT_ = (VOCAB_ + VTILE_ - 1) // VTILE_
_PREC = jax.lax.Precision.DEFAULT


def _fused_kernel(x_ref, emb_hbm, w1_ref, b1_ref, w2_ref, b2_ref,
                  logits_ref, lse_ref, flat_ref, h_ref, m_ref, s_ref, sem):
    t = pl.program_id(0)

    @pl.when(t == 0)
    def _gather_fc1():
        copies = []
        for j in range(CTX2_):
            c = pltpu.make_async_copy(
                emb_hbm.at[pl.ds(x_ref[j], 1), :],
                flat_ref.at[:, pl.ds(j * EMB_, EMB_)],
                sem)
            c.start()
            copies.append(c)
        for c in copies:
            c.wait()
        p = jax.lax.dot_general(flat_ref[...], w1_ref[...],
                                (((1,), (1,)), ((), ())),
                                precision=_PREC,
                                preferred_element_type=jnp.float32)
        h_ref[...] = jnp.maximum(p + b1_ref[...], 0.0)

    h = h_ref[...]
    w = w2_ref[...]
    l = jax.lax.dot_general(h, w, (((1,), (1,)), ((), ())),
                            precision=_PREC,
                            preferred_element_type=jnp.float32)
    l = l + b2_ref[...]
    logits_ref[...] = l

    col = t * VTILE_ + jax.lax.broadcasted_iota(jnp.int32, (1, VTILE_), 1)
    lm = jnp.where(col < VOCAB_, l, -jnp.inf)
    tmax = jnp.max(lm)

    @pl.when(t == 0)
    def _init():
        m_ref[0, 0] = tmax
        s_ref[0, 0] = jnp.sum(jnp.exp(lm - tmax))

    @pl.when(t > 0)
    def _acc():
        m_old = m_ref[0, 0]
        m_new = jnp.maximum(m_old, tmax)
        s_ref[0, 0] = (s_ref[0, 0] * jnp.exp(m_old - m_new)
                       + jnp.sum(jnp.exp(lm - m_new)))
        m_ref[0, 0] = m_new

    @pl.when(t == pl.num_programs(0) - 1)
    def _fin():
        lse_ref[...] = jnp.full((1, 1), m_ref[0, 0] + jnp.log(s_ref[0, 0]),
                                dtype=jnp.float32)


def _norm_kernel(logits_ref, lse_ref, out_ref):
    out_ref[...] = logits_ref[...] - lse_ref[...]


@functools.partial(jax.jit, static_argnames=("interpret",))
def _run(x, emb, W1, b1, W2, b2, interpret=False):
    b1r = b1.reshape(1, HID_)
    b2r = b2.reshape(1, VOCAB_)

    logits, lse = pl.pallas_call(
        _fused_kernel,
        grid_spec=pltpu.PrefetchScalarGridSpec(
            num_scalar_prefetch=1,
            grid=(NT_,),
            in_specs=[
                pl.BlockSpec(memory_space=pltpu.MemorySpace.HBM),
                pl.BlockSpec((HID_, CTX2_ * EMB_), lambda t, xr: (0, 0)),
                pl.BlockSpec((1, HID_), lambda t, xr: (0, 0)),
                pl.BlockSpec((VTILE_, HID_), lambda t, xr: (t, 0)),
                pl.BlockSpec((1, VTILE_), lambda t, xr: (0, t)),
            ],
            out_specs=[
                pl.BlockSpec((1, VTILE_), lambda t, xr: (0, t)),
                pl.BlockSpec((1, 1), lambda t, xr: (0, 0)),
            ],
            scratch_shapes=[
                pltpu.VMEM((1, CTX2_ * EMB_), jnp.float32),
                pltpu.VMEM((1, HID_), jnp.float32),
                pltpu.SMEM((1, 1), jnp.float32),
                pltpu.SMEM((1, 1), jnp.float32),
                pltpu.SemaphoreType.DMA,
            ],
        ),
        out_shape=[
            jax.ShapeDtypeStruct((1, VOCAB_), jnp.float32),
            jax.ShapeDtypeStruct((1, 1), jnp.float32),
        ],
        interpret=interpret,
    )(x, emb, W1, b1r, W2, b2r)

    out = pl.pallas_call(
        _norm_kernel,
        interpret=interpret,
        out_shape=jax.ShapeDtypeStruct((1, VOCAB_), jnp.float32),
    )(logits, lse)

    return out


def kernel(x, emb, W1, b1, W2, b2):
    return _run(x, emb, W1, b1, W2, b2)


# speedup vs baseline: 1.0295x; 1.0295x over previous
import functools

import jax
import jax.numpy as jnp
from jax.experimental import pallas as pl
from jax.experimental.pallas import tpu as pltpu

VOCAB_ = 100000
EMB_ = 128
CTX2_ = 50
HID_ = 128
VTILE_ = 20480
NT_ = (VOCAB_ + VTILE_ - 1) // VTILE_
_PREC = jax.lax.Precision.DEFAULT


def _fused_kernel(x_ref, emb_hbm, w1_ref, b1_ref, w2_ref, b2_ref,
                  logits_ref, lse_ref, flat_ref, h_ref, m_ref, s_ref, sem):
    t = pl.program_id(0)

    @pl.when(t == 0)
    def _gather_fc1():
        copies = []
        for j in range(CTX2_):
            c = pltpu.make_async_copy(
                emb_hbm.at[pl.ds(x_ref[j], 1), :],
                flat_ref.at[:, pl.ds(j * EMB_, EMB_)],
                sem)
            c.start()
            copies.append(c)
        for c in copies:
            c.wait()
        p = jax.lax.dot_general(flat_ref[...], w1_ref[...],
                                (((1,), (1,)), ((), ())),
                                precision=_PREC,
                                preferred_element_type=jnp.float32)
        h_ref[...] = jnp.maximum(p + b1_ref[...], 0.0)

    h = h_ref[...]
    w = w2_ref[...]
    l = jax.lax.dot_general(h, w, (((1,), (1,)), ((), ())),
                            precision=_PREC,
                            preferred_element_type=jnp.float32)
    l = l + b2_ref[...]
    logits_ref[...] = l

    col = t * VTILE_ + jax.lax.broadcasted_iota(jnp.int32, (1, VTILE_), 1)
    lm = jnp.where(col < VOCAB_, l, -jnp.inf)
    tmax = jnp.max(lm)

    @pl.when(t == 0)
    def _init():
        m_ref[0, 0] = tmax
        s_ref[0, 0] = jnp.sum(jnp.exp(lm - tmax))

    @pl.when(t > 0)
    def _acc():
        m_old = m_ref[0, 0]
        m_new = jnp.maximum(m_old, tmax)
        s_ref[0, 0] = (s_ref[0, 0] * jnp.exp(m_old - m_new)
                       + jnp.sum(jnp.exp(lm - m_new)))
        m_ref[0, 0] = m_new

    @pl.when(t == pl.num_programs(0) - 1)
    def _fin():
        lse_ref[...] = jnp.full((1, 1), m_ref[0, 0] + jnp.log(s_ref[0, 0]),
                                dtype=jnp.float32)


def _norm_kernel(logits_ref, lse_ref, out_ref):
    out_ref[...] = logits_ref[...] - lse_ref[...]


@functools.partial(jax.jit, static_argnames=("interpret",))
def _run(x, emb, W1, b1, W2, b2, interpret=False):
    b1r = b1.reshape(1, HID_)
    b2r = b2.reshape(1, VOCAB_)

    logits, lse = pl.pallas_call(
        _fused_kernel,
        grid_spec=pltpu.PrefetchScalarGridSpec(
            num_scalar_prefetch=1,
            grid=(NT_,),
            in_specs=[
                pl.BlockSpec(memory_space=pltpu.MemorySpace.HBM),
                pl.BlockSpec((HID_, CTX2_ * EMB_), lambda t, xr: (0, 0)),
                pl.BlockSpec((1, HID_), lambda t, xr: (0, 0)),
                pl.BlockSpec((VTILE_, HID_), lambda t, xr: (t, 0)),
                pl.BlockSpec((1, VTILE_), lambda t, xr: (0, t)),
            ],
            out_specs=[
                pl.BlockSpec((1, VTILE_), lambda t, xr: (0, t)),
                pl.BlockSpec((1, 1), lambda t, xr: (0, 0)),
            ],
            scratch_shapes=[
                pltpu.VMEM((1, CTX2_ * EMB_), jnp.float32),
                pltpu.VMEM((1, HID_), jnp.float32),
                pltpu.SMEM((1, 1), jnp.float32),
                pltpu.SMEM((1, 1), jnp.float32),
                pltpu.SemaphoreType.DMA,
            ],
        ),
        out_shape=[
            jax.ShapeDtypeStruct((1, VOCAB_), jnp.float32),
            jax.ShapeDtypeStruct((1, 1), jnp.float32),
        ],
        interpret=interpret,
    )(x, emb, W1, b1r, W2, b2r)

    out = pl.pallas_call(
        _norm_kernel,
        interpret=interpret,
        input_output_aliases={0: 0},
        out_shape=jax.ShapeDtypeStruct((1, VOCAB_), jnp.float32),
    )(logits, lse)

    return out


def kernel(x, emb, W1, b1, W2, b2):
    return _run(x, emb, W1, b1, W2, b2)
